# baseline (speedup 1.0000x reference)
import functools

import jax
import jax.numpy as jnp
from jax import lax
from jax.experimental import pallas as pl
from jax.experimental.pallas import tpu as pltpu
from jax.experimental.pallas import tpu_sc as plsc

_INV_T = 10.0
_L = 16
_PAD_VAL = -6.25e28


@functools.lru_cache(maxsize=None)
def _make_sc_scores(B, N, D, NC, NS):
    NW = NC * NS
    assert B % NW == 0 and D == 4 * _L
    bw = B // NW
    assert bw % _L == 0
    n_full = N // _L
    n_tail = N % _L
    mesh = plsc.VectorSubcoreMesh(
        core_axis_name="c", subcore_axis_name="s",
        num_cores=NC, num_subcores=NS)

    def body(users_hbm, pos_hbm, neg_hbm, uemb_hbm, iemb_hbm, out_hbm,
             u_idx, p_idx, u_rows, p_rows,
             n_idx0, n_idx1, n_rows0, n_rows1,
             tbuf, tbuf2, sbuf, pbuf, res, sem0, sem1):
        wid = lax.axis_index("s") * NC + lax.axis_index("c")
        base = wid * bw
        lane = lax.iota(jnp.int32, _L)
        lane_l = lane * _L

        def treduce(buf):
            acc = plsc.load_gather(buf, [lane_l])
            for c in range(1, _L):
                acc = acc + plsc.load_gather(buf, [lane_l + c])
            return acc

        pltpu.sync_copy(users_hbm.at[pl.ds(base, bw)], u_idx)
        pltpu.sync_copy(pos_hbm.at[pl.ds(base, bw)], p_idx)
        pltpu.async_copy(uemb_hbm.at[u_idx], u_rows, sem0).wait()
        pltpu.async_copy(iemb_hbm.at[p_idx], p_rows, sem0).wait()

        if n_tail:
            for r in range(n_tail, _L):
                tbuf2[pl.ds(r * _L, _L)] = jnp.full((_L,), _PAD_VAL,
                                                    jnp.float32)

        n_idx = (n_idx0, n_idx1)
        n_rows = (n_rows0, n_rows1)
        sems = (sem0, sem1)

        def fetch(b, slot):
            pltpu.sync_copy(neg_hbm.at[base + b], n_idx[slot])
            pltpu.async_copy(iemb_hbm.at[n_idx[slot]], n_rows[slot],
                             sems[slot])

        def compute(b, slot):
            rows = n_rows[slot]
            u0 = u_rows[b, pl.ds(0, _L)]
            u1 = u_rows[b, pl.ds(_L, _L)]
            u2 = u_rows[b, pl.ds(2 * _L, _L)]
            u3 = u_rows[b, pl.ds(3 * _L, _L)]

            def dot_to(buf, r, j):
                t = (u0 * rows[j, pl.ds(0, _L)]
                     + u1 * rows[j, pl.ds(_L, _L)]
                     + u2 * rows[j, pl.ds(2 * _L, _L)]
                     + u3 * rows[j, pl.ds(3 * _L, _L)])
                buf[pl.ds(r * _L, _L)] = t

            def gbody(g, sacc):
                for r in range(_L):
                    dot_to(tbuf, r, g * _L + r)
                return sacc + jnp.exp(treduce(tbuf) * _INV_T)

            sacc = lax.fori_loop(0, n_full, gbody,
                                 jnp.zeros((_L,), jnp.float32), unroll=False)
            if n_tail:
                for r in range(n_tail):
                    dot_to(tbuf2, r, n_full * _L + r)
                sacc = sacc + jnp.exp(treduce(tbuf2) * _INV_T)

            tp = (u0 * p_rows[b, pl.ds(0, _L)]
                  + u1 * p_rows[b, pl.ds(_L, _L)]
                  + u2 * p_rows[b, pl.ds(2 * _L, _L)]
                  + u3 * p_rows[b, pl.ds(3 * _L, _L)])
            bm = jnp.bitwise_and(b, _L - 1)
            sbuf[pl.ds(bm * _L, _L)] = sacc
            pbuf[pl.ds(bm * _L, _L)] = tp

            @pl.when(bm == _L - 1)
            def _fold():
                svec = treduce(sbuf)
                pe = jnp.exp(treduce(pbuf) * _INV_T)
                res[pl.ds(b - (_L - 1), _L)] = pe / (pe + svec)

        fetch(0, 0)

        def outer(i, _):
            for k in range(2):
                b = i * 2 + k
                slot = k

                @pl.when(b + 1 < bw)
                def _prefetch():
                    fetch(b + 1, 1 - slot)

                pltpu.make_async_copy(
                    iemb_hbm.at[n_idx[slot]], n_rows[slot],
                    sems[slot]).wait()
                compute(b, slot)
            return _

        lax.fori_loop(0, bw // 2, outer, 0, unroll=False)
        pltpu.sync_copy(res, out_hbm.at[pl.ds(base, bw)])

    return pl.kernel(
        body,
        out_type=jax.ShapeDtypeStruct((B,), jnp.float32),
        mesh=mesh,
        compiler_params=pltpu.CompilerParams(
            needs_layout_passes=False, use_tc_tiling_on_sc=False),
        scratch_types=[
            pltpu.VMEM((bw,), jnp.int32),
            pltpu.VMEM((bw,), jnp.int32),
            pltpu.VMEM((bw, D), jnp.float32),
            pltpu.VMEM((bw, D), jnp.float32),
            pltpu.VMEM((N,), jnp.int32),
            pltpu.VMEM((N,), jnp.int32),
            pltpu.VMEM((N, D), jnp.float32),
            pltpu.VMEM((N, D), jnp.float32),
            pltpu.VMEM((_L * _L,), jnp.float32),
            pltpu.VMEM((_L * _L,), jnp.float32),
            pltpu.VMEM((_L * _L,), jnp.float32),
            pltpu.VMEM((_L * _L,), jnp.float32),
            pltpu.VMEM((bw,), jnp.float32),
            pltpu.SemaphoreType.DMA,
            pltpu.SemaphoreType.DMA,
        ],
    )


def _loss_body(r_ref, o_ref):
    o_ref[0, 0] = -jnp.mean(jnp.log(r_ref[...]))


@functools.lru_cache(maxsize=None)
def _make_tc_loss(rows, cols):
    return pl.pallas_call(
        _loss_body,
        out_shape=jax.ShapeDtypeStruct((1, 1), jnp.float32),
        out_specs=pl.BlockSpec(memory_space=pltpu.SMEM),
    )


def kernel(users, positives, negatives, user_emb, item_emb, epoch):
    B = users.shape[0]
    N = negatives.shape[1]
    D = user_emb.shape[1]
    info = plsc.get_sparse_core_info()
    sc = _make_sc_scores(B, N, D, info.num_cores, info.num_subcores)
    r = sc(users.astype(jnp.int32),
           positives.reshape(-1).astype(jnp.int32),
           negatives.astype(jnp.int32),
           user_emb, item_emb)
    loss = _make_tc_loss(B // 128, 128)(r.reshape(B // 128, 128))
    return loss[0, 0]
